# baseline (speedup 1.0000x reference)
import functools

import jax
import jax.numpy as jnp
from jax import lax
from jax.experimental import pallas as pl
from jax.experimental.pallas import tpu as pltpu
from jax.experimental.pallas import tpu_sc as plsc

VOCAB = 32000
CHAR_VOCAB = 6000
MAX_CHARS = 16
D_CHAR = 128
D_MODEL = 768

N_TOK = 4 * 4096
N_WORKERS = 32
TOK_PER_W = N_TOK // N_WORKERS
GRP = 8
N_GRP = TOK_PER_W // GRP
AUG = 160
AUX_W = 128


def _sc_pool(tok_hbm, aux_hbm, table_hbm, out_hbm,
             tok_v, aux_v, gidx, rows, out_v, sem):
    wid = lax.axis_index("s") * 2 + lax.axis_index("c")
    base = wid * TOK_PER_W
    iota = lax.iota(jnp.int32, 16)

    pltpu.sync_copy(tok_hbm.at[pl.ds(base, TOK_PER_W)], tok_v)
    for c in range(TOK_PER_W // 128):
        pltpu.async_copy(
            aux_hbm.at[tok_v.at[pl.ds(c * 128, 128)]],
            aux_v.at[pl.ds(c * 128, 128)], sem).wait()

    def group(g, carry):
        for t in range(GRP):
            tt = g * GRP + t
            cids = aux_v[tt, pl.ds(0, 16)]
            meta = aux_v[tt, pl.ds(16, 16)]
            ln = meta[0]
            gidx[pl.ds(t * 16, 16)] = jnp.where(iota < ln, cids, CHAR_VOCAB)
        pltpu.async_copy(table_hbm.at[gidx], rows, sem).wait()
        for t in range(GRP):
            tt = g * GRP + t
            meta = aux_v[tt, pl.ds(16, 16)]
            inv = lax.bitcast_convert_type(meta, jnp.float32)[3]
            for d in range(D_CHAR // 16):
                acc = rows[t * 16, pl.ds(d * 16, 16)]
                for j in range(1, MAX_CHARS):
                    acc = acc + rows[t * 16 + j, pl.ds(d * 16, 16)]
                out_v[t, pl.ds(d * 16, 16)] = acc * inv
            flag = meta[1]
            cont = meta[2]
            onehot = jnp.where((iota == flag) | (iota == cont + 3),
                               jnp.float32(1.0), jnp.float32(0.0))
            out_v[t, pl.ds(128, 16)] = onehot
            out_v[t, pl.ds(144, 16)] = jnp.zeros((16,), jnp.float32)
        pltpu.sync_copy(out_v, out_hbm.at[pl.ds(base + g * GRP, GRP)])
        return carry

    lax.fori_loop(0, N_GRP, group, 0)


_sc_pool_call = functools.partial(
    pl.kernel,
    out_type=jax.ShapeDtypeStruct((N_TOK, AUG), jnp.float32),
    mesh=plsc.VectorSubcoreMesh(core_axis_name="c", subcore_axis_name="s"),
    scratch_types=[
        pltpu.VMEM((TOK_PER_W,), jnp.int32),
        pltpu.VMEM((TOK_PER_W, AUX_W), jnp.int32),
        pltpu.VMEM((GRP * 16,), jnp.int32),
        pltpu.VMEM((GRP * 16, D_CHAR), jnp.float32),
        pltpu.VMEM((GRP, AUG), jnp.float32),
        pltpu.SemaphoreType.DMA,
    ],
)(_sc_pool)


def _mm_body(x_ref, w_ref, o_ref):
    o_ref[...] = jnp.dot(x_ref[...], w_ref[...],
                         preferred_element_type=jnp.float32)


def _project(pooled_aug, w_aug):
    bm = 256
    return pl.pallas_call(
        _mm_body,
        grid=(N_TOK // bm,),
        in_specs=[
            pl.BlockSpec((bm, AUG), lambda i: (i, 0)),
            pl.BlockSpec((AUG, D_MODEL), lambda i: (0, 0)),
        ],
        out_specs=pl.BlockSpec((bm, D_MODEL), lambda i: (i, 0)),
        out_shape=jax.ShapeDtypeStruct((N_TOK, D_MODEL), jnp.float32),
    )(pooled_aug, w_aug)


def kernel(token_ids, char_ids, char_lengths, char_table, proj_W,
           special_flags, special_emb, is_continuation, cont_emb):
    tok = token_ids.reshape(-1).astype(jnp.int32)
    nrows = char_ids.shape[0]
    inv_bits = jax.lax.bitcast_convert_type(
        1.0 / char_lengths.astype(jnp.float32), jnp.int32)
    aux = jnp.concatenate(
        [char_ids.astype(jnp.int32),
         char_lengths.astype(jnp.int32)[:, None],
         special_flags.astype(jnp.int32)[:, None],
         is_continuation.astype(jnp.int32)[:, None],
         inv_bits[:, None],
         jnp.zeros((nrows, AUX_W - MAX_CHARS - 4), jnp.int32)], axis=1)
    table_z = char_table.at[CHAR_VOCAB].set(0.0)
    w_aug = jnp.concatenate(
        [proj_W, special_emb, cont_emb,
         jnp.zeros((AUG - D_CHAR - 5, D_MODEL), jnp.float32)], axis=0)

    pooled_aug = _sc_pool_call(tok, aux, table_z)
    out = _project(pooled_aug, w_aug)
    return out.reshape(token_ids.shape[0], token_ids.shape[1], D_MODEL)


# speedup vs baseline: 1.0019x; 1.0019x over previous
import functools

import jax
import jax.numpy as jnp
from jax import lax
from jax.experimental import pallas as pl
from jax.experimental.pallas import tpu as pltpu
from jax.experimental.pallas import tpu_sc as plsc

VOCAB = 32000
CHAR_VOCAB = 6000
MAX_CHARS = 16
D_CHAR = 128
D_MODEL = 768

N_TOK = 4 * 4096
N_WORKERS = 32
TOK_PER_W = N_TOK // N_WORKERS
GRP = 2
NBUF = 8
SG = NBUF * GRP
N_SG = TOK_PER_W // SG
AUG = 160
LN_LANE = 5
AUX_W = 128


def _sc_pool(tok_hbm, aux_hbm, table_hbm, out_hbm,
             tok_v, aux_v, gidx, rows, out_v, sem, sem2):
    cid = lax.axis_index("c")
    sid = lax.axis_index("s")
    wid = sid * 2 + cid
    base = wid * TOK_PER_W
    iota = lax.iota(jnp.int32, 16)

    pltpu.sync_copy(tok_hbm.at[pl.ds(base, TOK_PER_W)], tok_v)
    cps = [pltpu.async_copy(
        aux_hbm.at[tok_v.at[pl.ds(c * 128, 128)]],
        aux_v.at[pl.ds(c * 128, 128)], sem2)
        for c in range(TOK_PER_W // 128)]
    for cp in cps:
        cp.wait()

    def super_group(i, carry):
        cps = []
        for b in range(NBUF):
            for q in range(GRP):
                t = b * GRP + q
                tt = i * SG + t
                cids = aux_v[tt, pl.ds(0, 16)]
                ln = aux_v[tt, pl.ds(16, 16)][0]
                gidx[b][pl.ds(q * 16, 16)] = jnp.where(
                    iota < ln, cids, CHAR_VOCAB)
            cps.append(pltpu.async_copy(
                table_hbm.at[gidx[b]], rows[b], sem))
        for b in range(NBUF):
            cps[b].wait()
            for q in range(GRP):
                t = b * GRP + q
                tt = i * SG + t
                meta = aux_v[tt, pl.ds(16, 16)]
                ln = meta[0]
                flag = meta[1]
                cont = meta[2]
                row = (b % 4) * GRP + q
                for k in range(D_CHAR // 16):
                    acc = rows[b][q * 16, pl.ds(k * 16, 16)]
                    for j in range(1, MAX_CHARS):
                        acc = acc + rows[b][q * 16 + j, pl.ds(k * 16, 16)]
                    out_v[row, pl.ds(16 * k, 16)] = acc
                lnf = ln.astype(jnp.float32)
                tail = jnp.where(
                    (iota == flag) | (iota == cont + 3) | (iota == LN_LANE),
                    lnf, jnp.float32(0.0))
                out_v[row, pl.ds(128, 16)] = tail
                out_v[row, pl.ds(144, 16)] = jnp.zeros((16,), jnp.float32)
            if b % 4 == 3:
                pltpu.sync_copy(
                    out_v,
                    out_hbm.at[pl.ds(base + i * SG + (b // 4) * 8, 8)])
        return carry

    lax.fori_loop(0, N_SG, super_group, 0)


_sc_pool_call = functools.partial(
    pl.kernel,
    out_type=jax.ShapeDtypeStruct((N_TOK, AUG), jnp.float32),
    mesh=plsc.VectorSubcoreMesh(core_axis_name="c", subcore_axis_name="s"),
    scratch_types=[
        pltpu.VMEM((TOK_PER_W,), jnp.int32),
        pltpu.VMEM((TOK_PER_W, AUX_W), jnp.int32),
        [pltpu.VMEM((GRP * 16,), jnp.int32) for _ in range(NBUF)],
        [pltpu.VMEM((GRP * 16, D_CHAR), jnp.float32) for _ in range(NBUF)],
        pltpu.VMEM((4 * GRP, AUG), jnp.float32),
        pltpu.SemaphoreType.DMA,
        pltpu.SemaphoreType.DMA,
    ],
)(_sc_pool)


def _mm_body(x_ref, w_ref, o_ref):
    x = x_ref[...]
    y = jnp.dot(x, w_ref[...], preferred_element_type=jnp.float32)
    o_ref[...] = y / x[:, 128 + LN_LANE:128 + LN_LANE + 1]


def _project(pooled_aug, w_aug):
    bm = 256
    return pl.pallas_call(
        _mm_body,
        grid=(N_TOK // bm,),
        in_specs=[
            pl.BlockSpec((bm, AUG), lambda i: (i, 0)),
            pl.BlockSpec((AUG, D_MODEL), lambda i: (0, 0)),
        ],
        out_specs=pl.BlockSpec((bm, D_MODEL), lambda i: (i, 0)),
        out_shape=jax.ShapeDtypeStruct((N_TOK, D_MODEL), jnp.float32),
    )(pooled_aug, w_aug)


def kernel(token_ids, char_ids, char_lengths, char_table, proj_W,
           special_flags, special_emb, is_continuation, cont_emb):
    tok = token_ids.reshape(-1).astype(jnp.int32)
    nrows = char_ids.shape[0]
    aux = jnp.concatenate(
        [char_ids.astype(jnp.int32),
         char_lengths.astype(jnp.int32)[:, None],
         special_flags.astype(jnp.int32)[:, None],
         is_continuation.astype(jnp.int32)[:, None],
         jnp.zeros((nrows, AUX_W - MAX_CHARS - 3), jnp.int32)], axis=1)
    table_z = char_table.at[CHAR_VOCAB].set(0.0)
    w_aug = jnp.concatenate(
        [proj_W, special_emb, cont_emb,
         jnp.zeros((AUG - D_CHAR - 5, D_MODEL), jnp.float32)], axis=0)

    pooled_aug = _sc_pool_call(tok, aux, table_z)
    out = _project(pooled_aug, w_aug)
    return out.reshape(token_ids.shape[0], token_ids.shape[1], D_MODEL)
